# Initial kernel scaffold; baseline (speedup 1.0000x reference)
import jax
import jax.numpy as jnp
from jax.experimental import pallas as pl


def kernel(x, k_w, k_b, q_w, q_b, v_w, v_b, r_w, r_b):
    raise NotImplementedError("write your pallas kernel here")


# speedup vs baseline: 1.2141x; 1.2141x over previous
import functools
import math

import jax
import jax.numpy as jnp
from jax.experimental import pallas as pl
from jax.experimental.pallas import tpu as pltpu


def _proj_body(x_ref, w_ref, b_ref, k_ref, q_ref, v_ref, *, rc):
    xb = x_ref[...].astype(jnp.bfloat16)
    kqv = jnp.dot(xb, w_ref[...], preferred_element_type=jnp.float32) + b_ref[...]
    kqv = kqv.astype(jnp.bfloat16)
    k_ref[...] = kqv[:, :rc]
    q_ref[...] = kqv[:, rc:2 * rc]
    v_ref[...] = kqv[:, 2 * rc:3 * rc]


def _attn_body(k_ref, q_ref, v_ref, rw_ref, rb_ref, o_ref, *, scale_inv):
    s = jax.lax.dot_general(
        k_ref[0], q_ref[0], (((1,), (1,)), ((), ())),
        preferred_element_type=jnp.float32)
    s = s * scale_inv
    m = jnp.max(s, axis=-1, keepdims=True)
    e = jnp.exp(s - m)
    den = jnp.sum(e, axis=-1, keepdims=True)
    p = (e / den).astype(jnp.bfloat16)
    o = jnp.dot(p, v_ref[0], preferred_element_type=jnp.float32)
    out = jnp.dot(o.astype(jnp.bfloat16), rw_ref[...],
                  preferred_element_type=jnp.float32) + rb_ref[...]
    o_ref[0] = out


def kernel(x, k_w, k_b, q_w, q_b, v_w, v_b, r_w, r_b):
    B, T, H, W, C = x.shape
    RC = k_w.shape[1]
    N = T * H * W
    scale_inv = 1.0 / math.sqrt(H * W * C)

    xf = x.reshape(B * N, C)
    wqkv = jnp.concatenate([k_w, q_w, v_w], axis=1).astype(jnp.bfloat16)
    bqkv = jnp.concatenate([k_b, q_b, v_b]).reshape(1, 3 * RC)

    BM = min(2048, B * N)
    kqv_shape = jax.ShapeDtypeStruct((B * N, RC), jnp.bfloat16)
    k_, q_, v_ = pl.pallas_call(
        functools.partial(_proj_body, rc=RC),
        grid=(B * N // BM,),
        in_specs=[
            pl.BlockSpec((BM, C), lambda i: (i, 0)),
            pl.BlockSpec((C, 3 * RC), lambda i: (0, 0)),
            pl.BlockSpec((1, 3 * RC), lambda i: (0, 0)),
        ],
        out_specs=[pl.BlockSpec((BM, RC), lambda i: (i, 0))] * 3,
        out_shape=[kqv_shape] * 3,
        compiler_params=pltpu.CompilerParams(
            dimension_semantics=("parallel",),
        ),
        name="qkv_proj",
    )(xf, wqkv, bqkv)

    kb = k_.reshape(B, N, RC)
    qb = q_.reshape(B, N, RC)
    vb = v_.reshape(B, N, RC)
    rw = r_w.astype(jnp.bfloat16)
    rb = r_b.reshape(1, C)

    BI = 512 if N % 512 == 0 else N
    out = pl.pallas_call(
        functools.partial(_attn_body, scale_inv=scale_inv),
        grid=(B, N // BI),
        in_specs=[
            pl.BlockSpec((1, BI, RC), lambda b, i: (b, i, 0)),
            pl.BlockSpec((1, N, RC), lambda b, i: (b, 0, 0)),
            pl.BlockSpec((1, N, RC), lambda b, i: (b, 0, 0)),
            pl.BlockSpec((RC, C), lambda b, i: (0, 0)),
            pl.BlockSpec((1, C), lambda b, i: (0, 0)),
        ],
        out_specs=pl.BlockSpec((1, BI, C), lambda b, i: (b, i, 0)),
        out_shape=jax.ShapeDtypeStruct((B, N, C), jnp.float32),
        compiler_params=pltpu.CompilerParams(
            dimension_semantics=("parallel", "arbitrary"),
            vmem_limit_bytes=50 * 1024 * 1024,
        ),
        name="attn_fused",
    )(kb, qb, vb, rw, rb)

    return out.reshape(B, T, H, W, C)


# speedup vs baseline: 1.5637x; 1.2879x over previous
import functools
import math

import jax
import jax.numpy as jnp
from jax.experimental import pallas as pl
from jax.experimental.pallas import tpu as pltpu


def _proj_body(x_ref, w_ref, b_ref, k_ref, qT_ref, vp_ref, *, rc):
    xb = x_ref[0].astype(jnp.bfloat16)
    kqv = jnp.dot(xb, w_ref[...], preferred_element_type=jnp.float32) + b_ref[...]
    kqv = kqv.astype(jnp.bfloat16)
    k_ref[0] = kqv[:, :rc]
    qT_ref[0] = kqv[:, rc:2 * rc].T
    ones = jnp.ones((kqv.shape[0], rc), jnp.bfloat16)
    vp_ref[0] = jnp.concatenate([kqv[:, 2 * rc:3 * rc], ones], axis=1)


def _attn_body(k_ref, qT_ref, vp_ref, rw_ref, rb_ref, o_ref, *, c2, rc):
    s = jnp.dot(k_ref[0], qT_ref[0], preferred_element_type=jnp.float32)
    m = jnp.max(s, axis=-1, keepdims=True)
    e = jnp.exp2((s - m) * c2).astype(jnp.bfloat16)
    o2 = jnp.dot(e, vp_ref[0], preferred_element_type=jnp.float32)
    o = (o2[:, :rc] / o2[:, rc:]).astype(jnp.bfloat16)
    out = jnp.dot(o, rw_ref[...], preferred_element_type=jnp.float32) + rb_ref[...]
    o_ref[0] = out


def kernel(x, k_w, k_b, q_w, q_b, v_w, v_b, r_w, r_b):
    B, T, H, W, C = x.shape
    RC = k_w.shape[1]
    N = T * H * W
    c2 = math.log2(math.e) / math.sqrt(H * W * C)

    xf = x.reshape(B, N, C)
    wqkv = jnp.concatenate([k_w, q_w, v_w], axis=1).astype(jnp.bfloat16)
    bqkv = jnp.concatenate([k_b, q_b, v_b]).reshape(1, 3 * RC)

    BM = min(2048, N)
    k_, qT, vp = pl.pallas_call(
        functools.partial(_proj_body, rc=RC),
        grid=(B, N // BM),
        in_specs=[
            pl.BlockSpec((1, BM, C), lambda b, j: (b, j, 0)),
            pl.BlockSpec((C, 3 * RC), lambda b, j: (0, 0)),
            pl.BlockSpec((1, 3 * RC), lambda b, j: (0, 0)),
        ],
        out_specs=[
            pl.BlockSpec((1, BM, RC), lambda b, j: (b, j, 0)),
            pl.BlockSpec((1, RC, BM), lambda b, j: (b, 0, j)),
            pl.BlockSpec((1, BM, 2 * RC), lambda b, j: (b, j, 0)),
        ],
        out_shape=[
            jax.ShapeDtypeStruct((B, N, RC), jnp.bfloat16),
            jax.ShapeDtypeStruct((B, RC, N), jnp.bfloat16),
            jax.ShapeDtypeStruct((B, N, 2 * RC), jnp.bfloat16),
        ],
        compiler_params=pltpu.CompilerParams(
            dimension_semantics=("parallel", "arbitrary"),
        ),
        name="qkv_proj",
    )(xf, wqkv, bqkv)

    rw = r_w.astype(jnp.bfloat16)
    rb = r_b.reshape(1, C)

    BI = 512 if N % 512 == 0 else N
    out = pl.pallas_call(
        functools.partial(_attn_body, c2=c2, rc=RC),
        grid=(B, N // BI),
        in_specs=[
            pl.BlockSpec((1, BI, RC), lambda b, i: (b, i, 0)),
            pl.BlockSpec((1, RC, N), lambda b, i: (b, 0, 0)),
            pl.BlockSpec((1, N, 2 * RC), lambda b, i: (b, 0, 0)),
            pl.BlockSpec((RC, C), lambda b, i: (0, 0)),
            pl.BlockSpec((1, C), lambda b, i: (0, 0)),
        ],
        out_specs=pl.BlockSpec((1, BI, C), lambda b, i: (b, i, 0)),
        out_shape=jax.ShapeDtypeStruct((B, N, C), jnp.float32),
        compiler_params=pltpu.CompilerParams(
            dimension_semantics=("parallel", "arbitrary"),
            vmem_limit_bytes=50 * 1024 * 1024,
        ),
        name="attn_fused",
    )(k_, qT, vp, rw, rb)

    return out.reshape(B, T, H, W, C)


# speedup vs baseline: 2.2651x; 1.4485x over previous
import functools
import math

import jax
import jax.numpy as jnp
from jax.experimental import pallas as pl
from jax.experimental.pallas import tpu as pltpu


def _proj_body(x_ref, w_ref, b_ref, k_ref, qT_ref, vp_ref, *, rc):
    xb = x_ref[0].astype(jnp.bfloat16)
    kqv = jnp.dot(xb, w_ref[...], preferred_element_type=jnp.float32) + b_ref[...]
    kqv = kqv.astype(jnp.bfloat16)
    k_ref[0] = kqv[:, :rc]
    qT_ref[0] = kqv[:, rc:2 * rc].T
    ones = jnp.ones((kqv.shape[0], rc), jnp.bfloat16)
    vp_ref[0] = jnp.concatenate([kqv[:, 2 * rc:3 * rc], ones], axis=1)


def _attn_body(k_ref, qT_ref, vp_ref, rw_ref, rb_ref, o_ref, *, c2, rc, n, ch):
    kh = k_ref[0]
    bi = kh.shape[0]
    acc = jnp.zeros((bi, 2 * rc), jnp.float32)
    m_run = jnp.full((bi, 1), -jnp.inf, jnp.bfloat16)
    for c in range(n // ch):
        sc = jnp.dot(kh, qT_ref[0, :, c * ch:(c + 1) * ch],
                     preferred_element_type=jnp.float32).astype(jnp.bfloat16)
        m_new = jnp.maximum(m_run, jnp.max(sc, axis=-1, keepdims=True))
        e = jnp.exp2((sc - m_new) * jnp.bfloat16(c2))
        corr = jnp.exp2((m_run - m_new).astype(jnp.float32) * c2)
        pv = jnp.dot(e, vp_ref[0, c * ch:(c + 1) * ch, :],
                     preferred_element_type=jnp.float32)
        acc = acc * corr + pv
        m_run = m_new
    o = (acc[:, :rc] / acc[:, rc:]).astype(jnp.bfloat16)
    out = jnp.dot(o, rw_ref[...], preferred_element_type=jnp.float32) + rb_ref[...]
    o_ref[0] = out


def kernel(x, k_w, k_b, q_w, q_b, v_w, v_b, r_w, r_b):
    B, T, H, W, C = x.shape
    RC = k_w.shape[1]
    N = T * H * W
    c2 = math.log2(math.e) / math.sqrt(H * W * C)

    xf = x.reshape(B, N, C)
    wqkv = jnp.concatenate([k_w, q_w, v_w], axis=1).astype(jnp.bfloat16)
    bqkv = jnp.concatenate([k_b, q_b, v_b]).reshape(1, 3 * RC)

    BM = min(2048, N)
    k_, qT, vp = pl.pallas_call(
        functools.partial(_proj_body, rc=RC),
        grid=(B, N // BM),
        in_specs=[
            pl.BlockSpec((1, BM, C), lambda b, j: (b, j, 0)),
            pl.BlockSpec((C, 3 * RC), lambda b, j: (0, 0)),
            pl.BlockSpec((1, 3 * RC), lambda b, j: (0, 0)),
        ],
        out_specs=[
            pl.BlockSpec((1, BM, RC), lambda b, j: (b, j, 0)),
            pl.BlockSpec((1, RC, BM), lambda b, j: (b, 0, j)),
            pl.BlockSpec((1, BM, 2 * RC), lambda b, j: (b, j, 0)),
        ],
        out_shape=[
            jax.ShapeDtypeStruct((B, N, RC), jnp.bfloat16),
            jax.ShapeDtypeStruct((B, RC, N), jnp.bfloat16),
            jax.ShapeDtypeStruct((B, N, 2 * RC), jnp.bfloat16),
        ],
        compiler_params=pltpu.CompilerParams(
            dimension_semantics=("parallel", "arbitrary"),
        ),
        name="qkv_proj",
    )(xf, wqkv, bqkv)

    rw = r_w.astype(jnp.bfloat16)
    rb = r_b.reshape(1, C)

    BI = 512 if N % 512 == 0 else N
    CH = 512 if N % 512 == 0 else N
    out = pl.pallas_call(
        functools.partial(_attn_body, c2=c2, rc=RC, n=N, ch=CH),
        grid=(B, N // BI),
        in_specs=[
            pl.BlockSpec((1, BI, RC), lambda b, i: (b, i, 0)),
            pl.BlockSpec((1, RC, N), lambda b, i: (b, 0, 0)),
            pl.BlockSpec((1, N, 2 * RC), lambda b, i: (b, 0, 0)),
            pl.BlockSpec((RC, C), lambda b, i: (0, 0)),
            pl.BlockSpec((1, C), lambda b, i: (0, 0)),
        ],
        out_specs=pl.BlockSpec((1, BI, C), lambda b, i: (b, i, 0)),
        out_shape=jax.ShapeDtypeStruct((B, N, C), jnp.float32),
        compiler_params=pltpu.CompilerParams(
            dimension_semantics=("parallel", "arbitrary"),
            vmem_limit_bytes=50 * 1024 * 1024,
        ),
        name="attn_fused",
    )(k_, qT, vp, rw, rb)

    return out.reshape(B, T, H, W, C)


# speedup vs baseline: 2.4167x; 1.0670x over previous
import functools
import math

import jax
import jax.numpy as jnp
from jax.experimental import pallas as pl
from jax.experimental.pallas import tpu as pltpu


def _proj_body(x_ref, w_ref, b_ref, k_ref, qT_ref, vp_ref, *, rc):
    xb = x_ref[0].astype(jnp.bfloat16)
    kqv = jnp.dot(xb, w_ref[...], preferred_element_type=jnp.float32) + b_ref[...]
    kqv = kqv.astype(jnp.bfloat16)
    k_ref[0] = kqv[:, :rc]
    qT_ref[0] = kqv[:, rc:2 * rc].T
    ones = jnp.ones((kqv.shape[0], rc), jnp.bfloat16)
    vp_ref[0] = jnp.concatenate([kqv[:, 2 * rc:3 * rc], ones], axis=1)


def _attn_body(k_ref, qT_ref, vp_ref, rw_ref, rb_ref, o_ref, *, c2, rc, n, ch):
    kh = k_ref[0]
    bi = kh.shape[0]
    acc = jnp.zeros((bi, 2 * rc), jnp.float32)
    m_run = jnp.full((bi, 1), -jnp.inf, jnp.bfloat16)
    for c in range(n // ch):
        sc = jnp.dot(kh, qT_ref[0, :, c * ch:(c + 1) * ch],
                     preferred_element_type=jnp.float32).astype(jnp.bfloat16)
        m_new = jnp.maximum(m_run, jnp.max(sc, axis=-1, keepdims=True))
        e = jnp.exp2((sc - m_new) * jnp.bfloat16(c2))
        corr = jnp.exp2((m_run - m_new).astype(jnp.float32) * c2)
        pv = jnp.dot(e, vp_ref[0, c * ch:(c + 1) * ch, :],
                     preferred_element_type=jnp.float32)
        acc = acc * corr + pv
        m_run = m_new
    o = (acc[:, :rc] / acc[:, rc:]).astype(jnp.bfloat16)
    out = jnp.dot(o, rw_ref[...], preferred_element_type=jnp.float32) + rb_ref[...]
    o_ref[0] = out


def kernel(x, k_w, k_b, q_w, q_b, v_w, v_b, r_w, r_b):
    B, T, H, W, C = x.shape
    RC = k_w.shape[1]
    N = T * H * W
    c2 = math.log2(math.e) / math.sqrt(H * W * C)

    xf = x.reshape(B, N, C)
    wqkv = jnp.concatenate([k_w, q_w, v_w], axis=1).astype(jnp.bfloat16)
    bqkv = jnp.concatenate([k_b, q_b, v_b]).reshape(1, 3 * RC)

    BM = min(2048, N)
    k_, qT, vp = pl.pallas_call(
        functools.partial(_proj_body, rc=RC),
        grid=(B, N // BM),
        in_specs=[
            pl.BlockSpec((1, BM, C), lambda b, j: (b, j, 0)),
            pl.BlockSpec((C, 3 * RC), lambda b, j: (0, 0)),
            pl.BlockSpec((1, 3 * RC), lambda b, j: (0, 0)),
        ],
        out_specs=[
            pl.BlockSpec((1, BM, RC), lambda b, j: (b, j, 0)),
            pl.BlockSpec((1, RC, BM), lambda b, j: (b, 0, j)),
            pl.BlockSpec((1, BM, 2 * RC), lambda b, j: (b, j, 0)),
        ],
        out_shape=[
            jax.ShapeDtypeStruct((B, N, RC), jnp.bfloat16),
            jax.ShapeDtypeStruct((B, RC, N), jnp.bfloat16),
            jax.ShapeDtypeStruct((B, N, 2 * RC), jnp.bfloat16),
        ],
        compiler_params=pltpu.CompilerParams(
            dimension_semantics=("parallel", "arbitrary"),
        ),
        name="qkv_proj",
    )(xf, wqkv, bqkv)

    rw = r_w.astype(jnp.bfloat16)
    rb = r_b.reshape(1, C)

    BI = 1024 if N % 1024 == 0 else N
    CH = 512 if N % 512 == 0 else N
    out = pl.pallas_call(
        functools.partial(_attn_body, c2=c2, rc=RC, n=N, ch=CH),
        grid=(B, N // BI),
        in_specs=[
            pl.BlockSpec((1, BI, RC), lambda b, i: (b, i, 0)),
            pl.BlockSpec((1, RC, N), lambda b, i: (b, 0, 0)),
            pl.BlockSpec((1, N, 2 * RC), lambda b, i: (b, 0, 0)),
            pl.BlockSpec((RC, C), lambda b, i: (0, 0)),
            pl.BlockSpec((1, C), lambda b, i: (0, 0)),
        ],
        out_specs=pl.BlockSpec((1, BI, C), lambda b, i: (b, i, 0)),
        out_shape=jax.ShapeDtypeStruct((B, N, C), jnp.float32),
        compiler_params=pltpu.CompilerParams(
            dimension_semantics=("parallel", "arbitrary"),
            vmem_limit_bytes=50 * 1024 * 1024,
        ),
        name="attn_fused",
    )(k_, qT, vp, rw, rb)

    return out.reshape(B, T, H, W, C)
